# baseline (speedup 1.0000x reference)
import functools

import jax
import jax.numpy as jnp
from jax import lax
from jax.experimental import pallas as pl
from jax.experimental.pallas import tpu as pltpu
from jax.experimental.pallas import tpu_sc as plsc

_N = 2048
_NPAIR = _N * (_N - 1) // 2
_CBUF = 16384

_TASKS = (
    (0, 1024, 1024, True, False), (0, 0, 1024, False, True),
    (1, 1024, 1024, True, False), (1, 0, 1024, False, True),
    (2, 1024, 1024, True, False), (2, 512, 512, False, True),
    (3, 1024, 1024, True, False), (3, 512, 512, False, True),
    (4, 1024, 1024, True, True),
    (5, 1024, 1024, True, True),
    (6, 1536, 512, True, True),
    (7, 1536, 512, True, True),
)
_CLS = {0: 16384, 1: 16384, 2: 12288, 3: 12288,
        4: 8192, 5: 8192, 6: 4096, 7: 2048}


def _partials(dist_mat, vec_planes, coef):
    mesh = plsc.VectorSubcoreMesh(core_axis_name="c", subcore_axis_name="s")

    @functools.partial(
        pl.kernel,
        out_type=jax.ShapeDtypeStruct((2, 64, 128), jnp.float32),
        mesh=mesh,
        scratch_types=[
            pltpu.VMEM((2, 8, 1024), jnp.float32),
            pltpu.VMEM((2, 8, 1024), jnp.float32),
            pltpu.VMEM((2, 8, 1024), jnp.float32),
            pltpu.VMEM((2, 8, 1024), jnp.float32),
            pltpu.VMEM((_CBUF + 32,), jnp.float32),
            pltpu.VMEM((_CBUF + 32,), jnp.float32),
            pltpu.VMEM((64, 128), jnp.float32),
            pltpu.VMEM((2, 8, 128), jnp.float32),
            pltpu.VMEM((8, 128), jnp.float32),
            pltpu.VMEM_SHARED((16, 64, 128), jnp.float32),
            pltpu.SemaphoreType.DMA,
            pltpu.SemaphoreType.DMA,
        ],
        compiler_params=pltpu.CompilerParams(
            needs_layout_passes=False, use_tc_tiling_on_sc=True),
    )
    def body(dist_hbm, vec_hbm, coef_hbm, out_hbm,
             d_buf, vx_buf, vy_buf, vz_buf, c_buf0, c_buf1, acc, tmp2,
             rsum, slabs, sem0, sem1):
        c = lax.axis_index("c")
        s = lax.axis_index("s")
        w = s * 2 + c
        lanes = lax.iota(jnp.int32, 16)
        zero16 = jnp.zeros((16,), jnp.float32)
        sems = (sem0, sem1)
        c_bufs = (c_buf0, c_buf1)

        def zrow(t, carry):
            acc[t >> 3, pl.ds((t & 7) * 16, 16)] = zero16
            return carry

        with jax.named_scope("ph_init"):
            lax.fori_loop(0, 64 * 8, zrow, 0)

        ginfo = {}

        def issue(t):
            kg, cs, wdt, first, _ = _TASKS[t]
            pb = t % 2
            sem = sems[t % 2]
            g = w + 32 * kg
            rows = pl.ds(pl.multiple_of(8 * g, 8), 8)
            cols = pl.ds(cs, wdt)
            dst = pl.ds(0, wdt)
            cps = [
                pltpu.async_copy(dist_hbm.at[rows, cols],
                                 d_buf.at[pb, :, dst], sem),
                pltpu.async_copy(vec_hbm.at[0, rows, cols],
                                 vx_buf.at[pb, :, dst], sem),
                pltpu.async_copy(vec_hbm.at[1, rows, cols],
                                 vy_buf.at[pb, :, dst], sem),
                pltpu.async_copy(vec_hbm.at[2, rows, cols],
                                 vz_buf.at[pb, :, dst], sem),
            ]
            if first:
                i0 = 8 * g
                cl = _CLS[kg]
                off0 = (i0 * (2 * _N - 1 - i0)) >> 1
                c0 = pl.multiple_of(jnp.minimum(off0 & -8, _NPAIR - cl), 8)
                shiftms = []
                for r in range(8):
                    i = i0 + r
                    off = (i * (2 * _N - 1 - i)) >> 1
                    shiftms.append(off - c0 - i - 1 + 24)
                ginfo[kg] = (i0, shiftms)
                cps.append(pltpu.async_copy(
                    coef_hbm.at[pl.ds(c0, cl)],
                    c_bufs[kg % 2].at[pl.ds(24, cl)], sem))
            return cps

        def make_chunk(pb, cs, kg, masked):
            i0, shiftms = ginfo[kg]
            cb = c_bufs[kg % 2]

            def chunk_math(T, carry):
                eas = list(carry[0:4])
                rs = list(carry[4:])
                base = T * 16
                bcs = pl.ds(base - cs, 16)
                cols6 = [zero16] * 6
                for r in range(8):
                    i = i0 + r
                    d = d_buf[pb, r, bcs]
                    cf = plsc.load_gather(cb, [lanes + (base + shiftms[r])])
                    inv = 1.0 / d
                    inv2 = inv * inv
                    inv3 = inv2 * inv
                    inv6 = inv3 * inv3
                    if masked:
                        m = ((lanes + base) > i) & (d <= 9.0)
                    else:
                        m = d <= 9.0
                    cm = jnp.where(m, cf, 0.0)
                    e = cm * inv6
                    eas[r % 4] = eas[r % 4] + e
                    wv = e * inv
                    fx = wv * vx_buf[pb, r, bcs]
                    fy = wv * vy_buf[pb, r, bcs]
                    fz = wv * vz_buf[pb, r, bcs]
                    p = r & 1
                    cols6[p] = cols6[p] + fx
                    cols6[2 + p] = cols6[2 + p] + fy
                    cols6[4 + p] = cols6[4 + p] + fz
                    rs[3 * r] = rs[3 * r] + fx
                    rs[3 * r + 1] = rs[3 * r + 1] + fy
                    rs[3 * r + 2] = rs[3 * r + 2] + fz
                pr = T >> 3
                pc = pl.ds((T & 7) * 16, 16)
                acc[pr, pc] = acc[pr, pc] - (cols6[0] + cols6[1])
                acc[16 + pr, pc] = acc[16 + pr, pc] - (cols6[2] + cols6[3])
                acc[32 + pr, pc] = acc[32 + pr, pc] - (cols6[4] + cols6[5])
                return tuple(eas) + tuple(rs)

            return chunk_math

        eas = (zero16,) * 4
        rs = (zero16,) * 24
        cps = {0: issue(0)}
        for t, task in enumerate(_TASKS):
            scope = jax.named_scope(f"ph_task{t}")
            scope.__enter__()
            kg, cs, wdt, first, last = task
            pb = t % 2
            if t + 1 < len(_TASKS):
                cps[t + 1] = issue(t + 1)
            for cp in cps.pop(t):
                cp.wait()
            i0, _ = ginfo[kg]
            carry = eas + rs
            if last:
                t_start = ((i0 + 1) & -16) >> 4
                carry = make_chunk(pb, cs, kg, True)(t_start, carry)
                carry = lax.fori_loop(t_start + 1, (cs + wdt) // 16,
                                      make_chunk(pb, cs, kg, False), carry)
            else:
                carry = lax.fori_loop(cs // 16, (cs + wdt) // 16,
                                      make_chunk(pb, cs, kg, False), carry)
            eas = tuple(carry[0:4])
            rs = tuple(carry[4:])
            if last:
                for r in range(8):
                    i = i0 + r
                    pr = i >> 7
                    pc = pl.ds((i >> 4 & 7) * 16, 16)
                    lm = lanes == (i & 15)
                    acc[pr, pc] = acc[pr, pc] + jnp.where(
                        lm, jnp.sum(rs[3 * r]), 0.0)
                    acc[16 + pr, pc] = acc[16 + pr, pc] + jnp.where(
                        lm, jnp.sum(rs[3 * r + 1]), 0.0)
                    acc[32 + pr, pc] = acc[32 + pr, pc] + jnp.where(
                        lm, jnp.sum(rs[3 * r + 2]), 0.0)
                rs = (zero16,) * 24
            scope.__exit__(None, None, None)

        def scale(t, carry):
            pr = t >> 3
            pc = pl.ds((t & 7) * 16, 16)
            acc[pr, pc] = acc[pr, pc] * -6.0
            return carry

        with jax.named_scope("ph_scale"):
            lax.fori_loop(0, 48 * 8, scale, 0)
        acc[48, pl.ds(0, 16)] = (acc[48, pl.ds(0, 16)]
                                 + (eas[0] + eas[1]) + (eas[2] + eas[3]))

        with jax.named_scope("ph_publish"):
            pltpu.sync_copy(acc, slabs.at[s])
            plsc.subcore_barrier()

        @pl.when(s < 8)
        def _reduce():
            seg = pl.ds(pl.multiple_of(8 * s, 8), 8)
            pend = {0: pltpu.async_copy(slabs.at[0, seg], tmp2.at[0], sem0)}
            for p in range(16):
                if p + 1 < 16:
                    pend[p + 1] = pltpu.async_copy(
                        slabs.at[p + 1, seg], tmp2.at[(p + 1) % 2],
                        sems[(p + 1) % 2])
                pend.pop(p).wait()

                def red(t, carry, p=p):
                    r = t >> 3
                    pc = pl.ds((t & 7) * 16, 16)
                    v = tmp2[p % 2, r, pc]
                    if p > 0:
                        v = v + rsum[r, pc]
                    rsum[r, pc] = v
                    return carry

                lax.fori_loop(0, 64, red, 0)
            pltpu.sync_copy(rsum, out_hbm.at[c, seg])

    return body(dist_mat, vec_planes, coef)


def kernel(dist_mat, vector_mat, forces_out, coord_idx, repulsion_B_coef,
           calc_energy=True, calc_forces=True):
    del coord_idx
    vec_planes = jnp.transpose(vector_mat, (2, 0, 1))
    out = _partials(dist_mat, vec_planes, repulsion_B_coef)
    tot = out[0] + out[1]
    forces = jnp.stack(
        [tot[0:16].reshape(_N), tot[16:32].reshape(_N),
         tot[32:48].reshape(_N)], axis=-1)
    energy = jnp.where(calc_energy, jnp.sum(tot[48, :16]), jnp.float32(0.0))
    forces_ret = jnp.where(calc_forces, forces_out + forces, forces_out)
    return energy, forces_ret


# speedup vs baseline: 1.0351x; 1.0351x over previous
import functools

import jax
import jax.numpy as jnp
from jax import lax
from jax.experimental import pallas as pl
from jax.experimental.pallas import tpu as pltpu
from jax.experimental.pallas import tpu_sc as plsc

_N = 2048
_NPAIR = _N * (_N - 1) // 2
_CBUF = 16384


def _partials(dist_mat, vec_planes, coef):
    mesh = plsc.VectorSubcoreMesh(core_axis_name="c", subcore_axis_name="s")

    @functools.partial(
        pl.kernel,
        out_type=jax.ShapeDtypeStruct((2, 64, 128), jnp.float32),
        mesh=mesh,
        scratch_types=[
            pltpu.VMEM((8, _N), jnp.float32),
            pltpu.VMEM((8, _N), jnp.float32),
            pltpu.VMEM((8, _N), jnp.float32),
            pltpu.VMEM((8, _N), jnp.float32),
            pltpu.VMEM((_CBUF + 32,), jnp.float32),
            pltpu.VMEM((64, 128), jnp.float32),
            pltpu.VMEM((16, 8, 128), jnp.float32),
            pltpu.VMEM((8, 128), jnp.float32),
            pltpu.VMEM_SHARED((16, 64, 128), jnp.float32),
            pltpu.SemaphoreType.DMA,
        ],
        compiler_params=pltpu.CompilerParams(
            needs_layout_passes=False, use_tc_tiling_on_sc=True),
    )
    def body(dist_hbm, vec_hbm, coef_hbm, out_hbm,
             d_buf, vx_buf, vy_buf, vz_buf, c_buf, acc, tmp16, rsum,
             slabs, sem):
        c = lax.axis_index("c")
        s = lax.axis_index("s")
        w = s * 2 + c
        lanes = lax.iota(jnp.int32, 16)
        zero16 = jnp.zeros((16,), jnp.float32)

        def zrow(t, carry):
            acc[t >> 3, pl.ds((t & 7) * 16, 16)] = zero16
            return carry

        lax.fori_loop(0, 64 * 8, zrow, 0)

        def group_body(kg, eacc_g):
            g = w + 32 * kg
            r0 = pl.multiple_of(8 * g, 8)
            i0 = 8 * g
            off0 = (i0 * (2 * _N - 1 - i0)) >> 1
            bsel = (_N - (i0 & -128) + 511) >> 9
            for kb in (1, 2, 3, 4):
                @pl.when(bsel == kb)
                def _issue(kb=kb):
                    wdt = 512 * kb
                    cs = _N - wdt
                    cl = 8 * wdt
                    rows = pl.ds(r0, 8)
                    cols = pl.ds(cs, wdt)
                    c0 = pl.multiple_of(
                        jnp.minimum(off0 & -8, _NPAIR - cl), 8)
                    cps = [
                        pltpu.async_copy(dist_hbm.at[rows, cols],
                                         d_buf.at[:, cols], sem),
                        pltpu.async_copy(vec_hbm.at[0, rows, cols],
                                         vx_buf.at[:, cols], sem),
                        pltpu.async_copy(vec_hbm.at[1, rows, cols],
                                         vy_buf.at[:, cols], sem),
                        pltpu.async_copy(vec_hbm.at[2, rows, cols],
                                         vz_buf.at[:, cols], sem),
                        pltpu.async_copy(coef_hbm.at[pl.ds(c0, cl)],
                                         c_buf.at[pl.ds(24, cl)], sem),
                    ]
                    for cp in cps:
                        cp.wait()
            c0 = pl.multiple_of(
                jnp.minimum(off0 & -8, _NPAIR - 8 * 512 * bsel), 8)
            shiftms = []
            for r in range(8):
                i = i0 + r
                off = (i * (2 * _N - 1 - i)) >> 1
                shiftms.append(off - c0 - i - 1 + 24)
            t_start = ((i0 + 1) & -16) >> 4

            def chunk_math(T, carry, masked):
                eas = list(carry[0:4])
                rs = list(carry[4:])
                base = T * 16
                cs = pl.ds(base, 16)
                cols = [zero16] * 6
                for r in range(8):
                    i = i0 + r
                    d = d_buf[r, cs]
                    cf = plsc.load_gather(c_buf, [lanes + (base + shiftms[r])])
                    inv = 1.0 / d
                    inv2 = inv * inv
                    inv3 = inv2 * inv
                    inv6 = inv3 * inv3
                    if masked:
                        m = ((lanes + base) > i) & (d <= 9.0)
                    else:
                        m = d <= 9.0
                    cm = jnp.where(m, cf, 0.0)
                    e = cm * inv6
                    eas[r % 4] = eas[r % 4] + e
                    wv = e * inv
                    fx = wv * vx_buf[r, cs]
                    fy = wv * vy_buf[r, cs]
                    fz = wv * vz_buf[r, cs]
                    p = r & 1
                    cols[p] = cols[p] + fx
                    cols[2 + p] = cols[2 + p] + fy
                    cols[4 + p] = cols[4 + p] + fz
                    rs[3 * r] = rs[3 * r] + fx
                    rs[3 * r + 1] = rs[3 * r + 1] + fy
                    rs[3 * r + 2] = rs[3 * r + 2] + fz
                pr = T >> 3
                pc = pl.ds((T & 7) * 16, 16)
                acc[pr, pc] = acc[pr, pc] - (cols[0] + cols[1])
                acc[16 + pr, pc] = acc[16 + pr, pc] - (cols[2] + cols[3])
                acc[32 + pr, pc] = acc[32 + pr, pc] - (cols[4] + cols[5])
                return tuple(eas) + tuple(rs)

            init = tuple(eacc_g) + (zero16,) * 24
            carry1 = chunk_math(t_start, init, masked=True)
            res = lax.fori_loop(
                t_start + 1, _N // 16,
                lambda T, cr: chunk_math(T, cr, masked=False), carry1)
            for r in range(8):
                i = i0 + r
                pr = i >> 7
                pc = pl.ds((i >> 4 & 7) * 16, 16)
                lm = lanes == (i & 15)
                acc[pr, pc] = acc[pr, pc] + jnp.where(
                    lm, jnp.sum(res[4 + 3 * r]), 0.0)
                acc[16 + pr, pc] = acc[16 + pr, pc] + jnp.where(
                    lm, jnp.sum(res[5 + 3 * r]), 0.0)
                acc[32 + pr, pc] = acc[32 + pr, pc] + jnp.where(
                    lm, jnp.sum(res[6 + 3 * r]), 0.0)
            return res[0:4]

        eaccs = lax.fori_loop(0, 8, group_body, (zero16,) * 4)

        def scale(t, carry):
            pr = t >> 3
            pc = pl.ds((t & 7) * 16, 16)
            acc[pr, pc] = acc[pr, pc] * -6.0
            return carry

        lax.fori_loop(0, 48 * 8, scale, 0)
        acc[48, pl.ds(0, 16)] = (acc[48, pl.ds(0, 16)]
                                 + (eaccs[0] + eaccs[1])
                                 + (eaccs[2] + eaccs[3]))

        pltpu.sync_copy(acc, slabs.at[s])
        plsc.subcore_barrier()

        @pl.when(s < 8)
        def _reduce():
            seg = pl.ds(pl.multiple_of(8 * s, 8), 8)
            cps = [pltpu.async_copy(slabs.at[p, seg], tmp16.at[p], sem)
                   for p in range(16)]
            for cp in cps:
                cp.wait()

            def red(t, carry):
                r = t >> 3
                pc = pl.ds((t & 7) * 16, 16)
                v = tmp16[0, r, pc]
                for p in range(1, 16):
                    v = v + tmp16[p, r, pc]
                rsum[r, pc] = v
                return carry

            lax.fori_loop(0, 64, red, 0)
            pltpu.sync_copy(rsum, out_hbm.at[c, seg])

    return body(dist_mat, vec_planes, coef)


def kernel(dist_mat, vector_mat, forces_out, coord_idx, repulsion_B_coef,
           calc_energy=True, calc_forces=True):
    del coord_idx
    vec_planes = jnp.transpose(vector_mat, (2, 0, 1))
    out = _partials(dist_mat, vec_planes, repulsion_B_coef)
    tot = out[0] + out[1]
    forces = jnp.stack(
        [tot[0:16].reshape(_N), tot[16:32].reshape(_N),
         tot[32:48].reshape(_N)], axis=-1)
    energy = jnp.where(calc_energy, jnp.sum(tot[48, :16]), jnp.float32(0.0))
    forces_ret = jnp.where(calc_forces, forces_out + forces, forces_out)
    return energy, forces_ret
